# Initial kernel scaffold; baseline (speedup 1.0000x reference)
import functools

import jax
import jax.numpy as jnp
from jax import lax
from jax.experimental import pallas as pl
from jax.experimental.pallas import tpu as pltpu
from jax.experimental.pallas import tpu_sc as plsc

RC = 5.2
NSHF = 16
LANES = 16


def _make_sc_distances(B, E, A):
    NW = 32
    CH = E // NW
    NIT = (CH + LANES - 1) // LANES
    CHP = NIT * LANES

    mesh = plsc.VectorSubcoreMesh(core_axis_name="c", subcore_axis_name="s")

    @functools.partial(
        pl.kernel,
        mesh=mesh,
        out_type=jax.ShapeDtypeStruct((B, E), jnp.float32),
        scratch_types=[
            pltpu.VMEM((CH, 2), jnp.int32),
            pltpu.VMEM((A, 3), jnp.float32),
            pltpu.VMEM((CHP,), jnp.float32),
        ],
    )
    def sck(conn_hbm, coords_hbm, d2_hbm, conn_v, coords_v, d2_v):
        cid = lax.axis_index("c")
        sid = lax.axis_index("s")
        w = sid * 2 + cid
        base_e = w * CH

        iota = lax.broadcasted_iota(jnp.int32, (LANES,), 0)
        c0 = jnp.zeros((LANES,), jnp.int32)
        c1 = jnp.full((LANES,), 1, jnp.int32)
        c2 = jnp.full((LANES,), 2, jnp.int32)

        def batch_body(b, carry):
            pltpu.sync_copy(conn_hbm.at[b, pl.ds(base_e, CH)], conn_v)
            pltpu.sync_copy(coords_hbm.at[b], coords_v)

            def inner(i, carry2):
                e = jnp.minimum(iota + i * LANES, CH - 1)
                ia = plsc.load_gather(conn_v, [e, c0])
                idn = plsc.load_gather(conn_v, [e, c1])
                ia = jnp.clip(ia, 0, A - 1)
                idn = jnp.clip(idn, 0, A - 1)
                ax = plsc.load_gather(coords_v, [ia, c0])
                ay = plsc.load_gather(coords_v, [ia, c1])
                az = plsc.load_gather(coords_v, [ia, c2])
                dx = ax - plsc.load_gather(coords_v, [idn, c0])
                dy = ay - plsc.load_gather(coords_v, [idn, c1])
                dz = az - plsc.load_gather(coords_v, [idn, c2])
                d2_v[pl.ds(i * LANES, LANES)] = dx * dx + dy * dy + dz * dz
                return carry2

            lax.fori_loop(0, NIT, inner, 0)
            pltpu.sync_copy(d2_v.at[pl.ds(0, CH)],
                            d2_hbm.at[b, pl.ds(base_e, CH)])
            return carry

        lax.fori_loop(0, B, batch_body, 0)

    return sck


def _tc_body(d2_ref, shf_ref, eta_ref, out_ref):
    EB = d2_ref.shape[1]
    eta = eta_ref[0, 0]
    d2 = d2_ref[...]
    d = jnp.sqrt(d2)
    fc = jnp.where(d <= RC, 0.5 * jnp.cos(d * (jnp.pi / RC)) + 0.5, 0.0)
    a = fc * 0.25
    dd = jnp.broadcast_to(d, (NSHF, EB))
    aa = jnp.broadcast_to(a, (NSHF, EB))
    s = jnp.broadcast_to(shf_ref[...], (NSHF, EB))
    t = dd - s
    r = aa * jnp.exp(t * t * (-eta))
    i0 = lax.broadcasted_iota(jnp.int32, (NSHF, NSHF), 0)
    i1 = lax.broadcasted_iota(jnp.int32, (NSHF, NSHF), 1)
    eye = (i0 == i1).astype(jnp.float32)
    out_ref[0] = lax.dot_general(r, eye, (((0,), (0,)), ((), ())),
                                 preferred_element_type=jnp.float32)


def _radial_tc(d2, ShfR, EtaR):
    n_total = d2.size
    EB = 12800
    R = n_total // EB
    d2r = d2.reshape(R, EB)
    shf_col = ShfR.reshape(NSHF, 1).astype(jnp.float32)
    eta = EtaR.reshape(1, 1).astype(jnp.float32)
    out = pl.pallas_call(
        _tc_body,
        grid=(R,),
        in_specs=[
            pl.BlockSpec((1, EB), lambda i: (i, 0)),
            pl.BlockSpec((NSHF, 1), lambda i: (0, 0)),
            pl.BlockSpec((1, 1), lambda i: (0, 0)),
        ],
        out_specs=pl.BlockSpec((1, EB, NSHF), lambda i: (i, 0, 0)),
        out_shape=jax.ShapeDtypeStruct((R, EB, NSHF), jnp.float32),
    )(d2r, shf_col, eta)
    return out


@jax.jit
def kernel(connectivity, coords, EtaR, ShfR):
    B, E, _ = connectivity.shape
    A = coords.shape[1]
    sck = _make_sc_distances(B, E, A)
    d2 = sck(connectivity, coords.astype(jnp.float32))
    y = _radial_tc(d2, ShfR, EtaR).reshape(B, E, NSHF)
    return (connectivity, y)


# speedup vs baseline: 3.2690x; 3.2690x over previous
import functools

import jax
import jax.numpy as jnp
from jax import lax
from jax.experimental import pallas as pl
from jax.experimental.pallas import tpu as pltpu
from jax.experimental.pallas import tpu_sc as plsc

RC = 5.2
NSHF = 16
LANES = 16
NW = 32


def _make_sc_distances(B, E, A):
    CH = E // NW
    NIT = (CH + LANES - 1) // LANES

    mesh = plsc.VectorSubcoreMesh(core_axis_name="c", subcore_axis_name="s")

    @functools.partial(
        pl.kernel,
        mesh=mesh,
        out_type=jax.ShapeDtypeStruct((NW, B, CH), jnp.float32),
        scratch_types=[
            pltpu.VMEM((CH, 2), jnp.int32),
            pltpu.VMEM((A, 3), jnp.float32),
            pltpu.VMEM((B, CH), jnp.float32),
        ],
        compiler_params=pltpu.CompilerParams(use_tc_tiling_on_sc=False,
                                             needs_layout_passes=False),
    )
    def sck(conn_hbm, coords_hbm, d2_hbm, conn_v, coords_v, d2_v):
        cid = lax.axis_index("c")
        sid = lax.axis_index("s")
        w = sid * 2 + cid
        base_e = w * CH

        iota = lax.broadcasted_iota(jnp.int32, (LANES,), 0)
        c0 = jnp.zeros((LANES,), jnp.int32)
        c1 = jnp.full((LANES,), 1, jnp.int32)
        c2 = jnp.full((LANES,), 2, jnp.int32)

        def batch_body(b, carry):
            pltpu.sync_copy(conn_hbm.at[b, pl.ds(base_e, CH)], conn_v)
            pltpu.sync_copy(coords_hbm.at[b], coords_v)

            def inner(i, carry2):
                base = jnp.minimum(i * LANES, CH - LANES)
                e = iota + base
                ia = plsc.load_gather(conn_v, [e, c0])
                idn = plsc.load_gather(conn_v, [e, c1])
                ia = jnp.clip(ia, 0, A - 1)
                idn = jnp.clip(idn, 0, A - 1)
                ax = plsc.load_gather(coords_v, [ia, c0])
                ay = plsc.load_gather(coords_v, [ia, c1])
                az = plsc.load_gather(coords_v, [ia, c2])
                dx = ax - plsc.load_gather(coords_v, [idn, c0])
                dy = ay - plsc.load_gather(coords_v, [idn, c1])
                dz = az - plsc.load_gather(coords_v, [idn, c2])
                d2_v[b, pl.ds(base, LANES)] = dx * dx + dy * dy + dz * dz
                return carry2

            lax.fori_loop(0, NIT, inner, 0)
            return carry

        lax.fori_loop(0, B, batch_body, 0)
        pltpu.sync_copy(d2_v, d2_hbm.at[w])

    return sck


def _tc_body(d2_ref, shf_ref, eta_ref, out_ref):
    EB = d2_ref.shape[-1]
    eta = eta_ref[0, 0]
    d2 = d2_ref[0, 0]
    d = jnp.sqrt(d2)
    fc = jnp.where(d <= RC, 0.5 * jnp.cos(d * (jnp.pi / RC)) + 0.5, 0.0)
    a = fc * 0.25
    dd = jnp.broadcast_to(d, (NSHF, EB))
    aa = jnp.broadcast_to(a, (NSHF, EB))
    s = jnp.broadcast_to(shf_ref[...], (NSHF, EB))
    t = dd - s
    r = aa * jnp.exp(t * t * (-eta))
    i0 = lax.broadcasted_iota(jnp.int32, (NSHF, NSHF), 0)
    i1 = lax.broadcasted_iota(jnp.int32, (NSHF, NSHF), 1)
    eye = (i0 == i1).astype(jnp.float32)
    out_ref[0] = lax.dot_general(r, eye, (((0,), (0,)), ((), ())),
                                 preferred_element_type=jnp.float32)


def _radial_tc(d2wmajor, ShfR, EtaR, B, E):
    CH = E // NW
    d2r = d2wmajor.reshape(NW, B, 1, CH)
    shf_col = ShfR.reshape(NSHF, 1).astype(jnp.float32)
    eta = EtaR.reshape(1, 1).astype(jnp.float32)
    out = pl.pallas_call(
        _tc_body,
        grid=(B, NW),
        in_specs=[
            pl.BlockSpec((1, 1, 1, CH), lambda b, w: (w, b, 0, 0)),
            pl.BlockSpec((NSHF, 1), lambda b, w: (0, 0)),
            pl.BlockSpec((1, 1), lambda b, w: (0, 0)),
        ],
        out_specs=pl.BlockSpec((1, CH, NSHF), lambda b, w: (b * NW + w, 0, 0)),
        out_shape=jax.ShapeDtypeStruct((B * NW, CH, NSHF), jnp.float32),
    )(d2r, shf_col, eta)
    return out


@jax.jit
def kernel(connectivity, coords, EtaR, ShfR):
    B, E, _ = connectivity.shape
    A = coords.shape[1]
    sck = _make_sc_distances(B, E, A)
    d2 = sck(connectivity, coords.astype(jnp.float32))
    y = _radial_tc(d2, ShfR, EtaR, B, E).reshape(B, E, NSHF)
    return (connectivity, y)


# speedup vs baseline: 4.9650x; 1.5188x over previous
import functools

import jax
import jax.numpy as jnp
from jax import lax
from jax.experimental import pallas as pl
from jax.experimental.pallas import tpu as pltpu
from jax.experimental.pallas import tpu_sc as plsc

RC = 5.2
NSHF = 16
LANES = 16
NW = 32


def _make_sc_distances(B, E, A):
    CH = E // NW
    NIT = (CH + LANES - 1) // LANES

    mesh = plsc.VectorSubcoreMesh(core_axis_name="c", subcore_axis_name="s")

    @functools.partial(
        pl.kernel,
        mesh=mesh,
        out_type=jax.ShapeDtypeStruct((B * E,), jnp.float32),
        scratch_types=[
            pltpu.VMEM((2 * CH,), jnp.int32),
            pltpu.VMEM((3 * A,), jnp.float32),
            pltpu.VMEM((CH,), jnp.float32),
        ],
        compiler_params=pltpu.CompilerParams(use_tc_tiling_on_sc=False,
                                             needs_layout_passes=False),
    )
    def sck(conn_hbm, coords_hbm, d2_hbm, conn_v, coords_v, d2_v):
        cid = lax.axis_index("c")
        sid = lax.axis_index("s")
        w = sid * 2 + cid

        iota = lax.broadcasted_iota(jnp.int32, (LANES,), 0)

        def batch_body(b, carry):
            pltpu.sync_copy(conn_hbm.at[pl.ds(b * 2 * E + w * 2 * CH, 2 * CH)],
                            conn_v)
            pltpu.sync_copy(coords_hbm.at[pl.ds(b * 3 * A, 3 * A)], coords_v)

            def inner(i, carry2):
                base = jnp.minimum(i * LANES, CH - LANES)
                e2 = (iota + base) * 2
                ia = plsc.load_gather(conn_v, [e2])
                idn = plsc.load_gather(conn_v, [e2 + 1])
                ia3 = jnp.clip(ia, 0, A - 1) * 3
                idn3 = jnp.clip(idn, 0, A - 1) * 3
                ax = plsc.load_gather(coords_v, [ia3])
                ay = plsc.load_gather(coords_v, [ia3 + 1])
                az = plsc.load_gather(coords_v, [ia3 + 2])
                dx = ax - plsc.load_gather(coords_v, [idn3])
                dy = ay - plsc.load_gather(coords_v, [idn3 + 1])
                dz = az - plsc.load_gather(coords_v, [idn3 + 2])
                d2_v[pl.ds(base, LANES)] = dx * dx + dy * dy + dz * dz
                return carry2

            lax.fori_loop(0, NIT, inner, 0)
            pltpu.sync_copy(d2_v, d2_hbm.at[pl.ds(b * E + w * CH, CH)])
            return carry

        lax.fori_loop(0, B, batch_body, 0)

    return sck


def _tc_body(d2_ref, shf_ref, eta_ref, out_ref):
    EB = d2_ref.shape[-1]
    eta = eta_ref[0, 0]
    d2 = d2_ref[0]
    d = jnp.sqrt(d2)
    fc = jnp.where(d <= RC, 0.5 * jnp.cos(d * (jnp.pi / RC)) + 0.5, 0.0)
    a = fc * 0.25
    dd = jnp.broadcast_to(d, (NSHF, EB))
    aa = jnp.broadcast_to(a, (NSHF, EB))
    s = jnp.broadcast_to(shf_ref[...], (NSHF, EB))
    t = dd - s
    r = aa * jnp.exp(t * t * (-eta))
    i0 = lax.broadcasted_iota(jnp.int32, (NSHF, NSHF), 0)
    i1 = lax.broadcasted_iota(jnp.int32, (NSHF, NSHF), 1)
    eye = (i0 == i1).astype(jnp.float32)
    out_ref[0] = lax.dot_general(r, eye, (((0,), (0,)), ((), ())),
                                 preferred_element_type=jnp.float32)


def _radial_tc(d2flat, ShfR, EtaR):
    n = d2flat.size
    EB = 12800
    R = n // EB
    d2r = d2flat.reshape(R, 1, EB)
    shf_col = ShfR.reshape(NSHF, 1).astype(jnp.float32)
    eta = EtaR.reshape(1, 1).astype(jnp.float32)
    out = pl.pallas_call(
        _tc_body,
        grid=(R,),
        in_specs=[
            pl.BlockSpec((1, 1, EB), lambda i: (i, 0, 0)),
            pl.BlockSpec((NSHF, 1), lambda i: (0, 0)),
            pl.BlockSpec((1, 1), lambda i: (0, 0)),
        ],
        out_specs=pl.BlockSpec((1, EB, NSHF), lambda i: (i, 0, 0)),
        out_shape=jax.ShapeDtypeStruct((R, EB, NSHF), jnp.float32),
    )(d2r, shf_col, eta)
    return out


@jax.jit
def kernel(connectivity, coords, EtaR, ShfR):
    B, E, _ = connectivity.shape
    A = coords.shape[1]
    sck = _make_sc_distances(B, E, A)
    d2 = sck(connectivity.reshape(-1),
             coords.astype(jnp.float32).reshape(-1))
    y = _radial_tc(d2, ShfR, EtaR).reshape(B, E, NSHF)
    return (connectivity, y)


# speedup vs baseline: 20.5557x; 4.1401x over previous
import functools

import jax
import jax.numpy as jnp
from jax import lax
from jax.experimental import pallas as pl
from jax.experimental.pallas import tpu as pltpu
from jax.experimental.pallas import tpu_sc as plsc

RC = 5.2
NSHF = 16
LANES = 16
NW = 32


def _make_sc_distances(B, E, A):
    CH = E // NW
    NIT = (CH + LANES - 1) // LANES

    mesh = plsc.VectorSubcoreMesh(core_axis_name="c", subcore_axis_name="s")

    @functools.partial(
        pl.kernel,
        mesh=mesh,
        out_type=jax.ShapeDtypeStruct((B * E,), jnp.float32),
        scratch_types=[
            pltpu.VMEM((CH,), jnp.int32),
            pltpu.VMEM((CH,), jnp.int32),
            pltpu.VMEM((A,), jnp.float32),
            pltpu.VMEM((A,), jnp.float32),
            pltpu.VMEM((A,), jnp.float32),
            pltpu.VMEM((CH,), jnp.float32),
        ],
        compiler_params=pltpu.CompilerParams(use_tc_tiling_on_sc=False,
                                             needs_layout_passes=False),
    )
    def sck(ia_hbm, id_hbm, cx_hbm, cy_hbm, cz_hbm, d2_hbm,
            ia_v, id_v, cx_v, cy_v, cz_v, d2_v):
        cid = lax.axis_index("c")
        sid = lax.axis_index("s")
        w = sid * 2 + cid

        def batch_body(b, carry):
            base_g = b * E + w * CH
            pltpu.sync_copy(ia_hbm.at[pl.ds(base_g, CH)], ia_v)
            pltpu.sync_copy(id_hbm.at[pl.ds(base_g, CH)], id_v)
            pltpu.sync_copy(cx_hbm.at[pl.ds(b * A, A)], cx_v)
            pltpu.sync_copy(cy_hbm.at[pl.ds(b * A, A)], cy_v)
            pltpu.sync_copy(cz_hbm.at[pl.ds(b * A, A)], cz_v)

            def inner(i, carry2):
                base = jnp.minimum(i * LANES, CH - LANES)
                ia = jnp.clip(ia_v[pl.ds(base, LANES)], 0, A - 1)
                idn = jnp.clip(id_v[pl.ds(base, LANES)], 0, A - 1)
                dx = plsc.load_gather(cx_v, [ia]) - plsc.load_gather(cx_v, [idn])
                dy = plsc.load_gather(cy_v, [ia]) - plsc.load_gather(cy_v, [idn])
                dz = plsc.load_gather(cz_v, [ia]) - plsc.load_gather(cz_v, [idn])
                d2_v[pl.ds(base, LANES)] = dx * dx + dy * dy + dz * dz
                return carry2

            lax.fori_loop(0, NIT, inner, 0)
            pltpu.sync_copy(d2_v, d2_hbm.at[pl.ds(b * E + w * CH, CH)])
            return carry

        lax.fori_loop(0, B, batch_body, 0)

    return sck


def _tc_body(d2_ref, shf_ref, eta_ref, out_ref):
    EB = d2_ref.shape[-1]
    eta = eta_ref[0, 0]
    d2 = d2_ref[0]
    d = jnp.sqrt(d2)
    fc = jnp.where(d <= RC, 0.5 * jnp.cos(d * (jnp.pi / RC)) + 0.5, 0.0)
    a = fc * 0.25
    dd = jnp.broadcast_to(d, (NSHF, EB))
    aa = jnp.broadcast_to(a, (NSHF, EB))
    s = jnp.broadcast_to(shf_ref[...], (NSHF, EB))
    t = dd - s
    r = aa * jnp.exp(t * t * (-eta))
    i0 = lax.broadcasted_iota(jnp.int32, (NSHF, NSHF), 0)
    i1 = lax.broadcasted_iota(jnp.int32, (NSHF, NSHF), 1)
    eye = (i0 == i1).astype(jnp.float32)
    out_ref[0] = lax.dot_general(r, eye, (((0,), (0,)), ((), ())),
                                 preferred_element_type=jnp.float32)


def _radial_tc(d2flat, ShfR, EtaR):
    n = d2flat.size
    EB = 12800
    R = n // EB
    d2r = d2flat.reshape(R, 1, EB)
    shf_col = ShfR.reshape(NSHF, 1).astype(jnp.float32)
    eta = EtaR.reshape(1, 1).astype(jnp.float32)
    out = pl.pallas_call(
        _tc_body,
        grid=(R,),
        in_specs=[
            pl.BlockSpec((1, 1, EB), lambda i: (i, 0, 0)),
            pl.BlockSpec((NSHF, 1), lambda i: (0, 0)),
            pl.BlockSpec((1, 1), lambda i: (0, 0)),
        ],
        out_specs=pl.BlockSpec((1, EB, NSHF), lambda i: (i, 0, 0)),
        out_shape=jax.ShapeDtypeStruct((R, EB, NSHF), jnp.float32),
    )(d2r, shf_col, eta)
    return out


@jax.jit
def kernel(connectivity, coords, EtaR, ShfR):
    B, E, _ = connectivity.shape
    A = coords.shape[1]
    sck = _make_sc_distances(B, E, A)
    cf = coords.astype(jnp.float32)
    d2 = sck(connectivity[:, :, 0].reshape(-1),
             connectivity[:, :, 1].reshape(-1),
             cf[:, :, 0].reshape(-1),
             cf[:, :, 1].reshape(-1),
             cf[:, :, 2].reshape(-1))
    y = _radial_tc(d2, ShfR, EtaR).reshape(B, E, NSHF)
    return (connectivity, y)
